# baseline (speedup 1.0000x reference)
import jax
import jax.numpy as jnp
from jax import lax
from jax.experimental import pallas as pl
from jax.experimental.pallas import tpu as pltpu
from jax.experimental.pallas import tpu_sc as plsc

NUM_UNITS = 32
MAX_REL = 128
LQ = 2048
LK = 2048
NW = 32
ROWS_PER_W = LQ // NW
W_COLS = 2560
UT = NUM_UNITS // 8
JT = LK // 128
TAB_WORDS = (2 * MAX_REL + 1) * NUM_UNITS


def _sc_body(tab_hbm, out_hbm, tab_vmem, w_vmem, sem):
    c = lax.axis_index("c")
    s = lax.axis_index("s")
    wid = s * 2 + c
    phi = wid % 8
    q = wid // 8
    wof = (LK - 1 - 504) - phi - 512 * q

    pltpu.sync_copy(tab_hbm, tab_vmem)

    lanes = lax.iota(jnp.int32, 16)

    m_band0 = (LK - 1 - MAX_REL) - wof
    mb_lo = m_band0 // 16
    mb_hi = (m_band0 + 2 * MAX_REL + 1 + 15) // 16

    def build_u(u, _):
        ut, us = u // 8, u % 8
        t0 = plsc.load_gather(tab_vmem, [lanes * 0 + u])
        t1 = plsc.load_gather(tab_vmem, [lanes * 0 + (2 * MAX_REL * NUM_UNITS + u)])

        def fill_head(mb, _):
            w_vmem[ut, us, pl.ds(mb * 16, 16)] = t0
            return 0

        def build_band(mb, _):
            m = mb * 16
            k = wof + m - (LK - 1) + lanes
            cidx = jnp.clip(k, -MAX_REL, MAX_REL) + MAX_REL
            w_vmem[ut, us, pl.ds(m, 16)] = plsc.load_gather(
                tab_vmem, [cidx * NUM_UNITS + u])
            return 0

        def fill_tail(mb, _):
            w_vmem[ut, us, pl.ds(mb * 16, 16)] = t1
            return 0

        lax.fori_loop(0, mb_lo, fill_head, 0)
        lax.fori_loop(mb_lo, mb_hi, build_band, 0)
        lax.fori_loop(mb_hi, W_COLS // 16, fill_tail, 0)
        return 0

    lax.fori_loop(0, NUM_UNITS, build_u, 0)

    def drain_row():
        for _ in range(JT):
            pltpu.make_async_copy(
                w_vmem.at[:, :, pl.ds(0, 128)],
                out_hbm.at[0, :, 0], sem).wait()

    def row(t, _):
        @pl.when(t > 3)
        def _():
            drain_row()
        i = phi + 512 * q + 8 * t
        s0 = 8 * (ROWS_PER_W - 1 - t)
        for jt in range(JT):
            pltpu.async_copy(
                w_vmem.at[:, :, pl.ds(s0 + 128 * jt, 128)],
                out_hbm.at[i, :, jt], sem)
        return 0

    lax.fori_loop(0, ROWS_PER_W, row, 0)
    for _ in range(4):
        drain_row()


def kernel(x, embeddings_table):
    del x
    run = pl.kernel(
        _sc_body,
        out_type=jax.ShapeDtypeStruct((LQ, UT, JT, 8, 128), jnp.float32),
        mesh=plsc.VectorSubcoreMesh(core_axis_name="c", subcore_axis_name="s"),
        scratch_types=[
            pltpu.VMEM((TAB_WORDS,), jnp.float32),
            pltpu.VMEM((UT, 8, W_COLS), jnp.float32),
            pltpu.SemaphoreType.DMA,
        ],
        compiler_params=pltpu.CompilerParams(use_tc_tiling_on_sc=False,
                                             needs_layout_passes=False),
    )
    out5 = run(embeddings_table.reshape(-1))
    return out5.transpose(0, 2, 4, 1, 3).reshape(LQ, LK, NUM_UNITS)


# speedup vs baseline: 1.0032x; 1.0032x over previous
import jax
import jax.numpy as jnp
from jax import lax
from jax.experimental import pallas as pl
from jax.experimental.pallas import tpu as pltpu
from jax.experimental.pallas import tpu_sc as plsc

NUM_UNITS = 32
MAX_REL = 128
LQ = 2048
LK = 2048
NW = 32
ROWS_PER_W = LQ // NW
W_COLS = 2560
UT = NUM_UNITS // 8
JT = LK // 128
TAB_WORDS = (2 * MAX_REL + 1) * NUM_UNITS


def _sc_body(tab_hbm, out_hbm, tab_vmem, w_vmem, sem):
    c = lax.axis_index("c")
    s = lax.axis_index("s")
    wid = s * 2 + c
    phi = wid % 8
    q = wid // 8
    wof = (LK - 1 - 504) - phi - 512 * q

    pltpu.sync_copy(tab_hbm, tab_vmem)

    lanes = lax.iota(jnp.int32, 16)

    m_band0 = (LK - 1 - MAX_REL) - wof
    mb_lo = m_band0 // 16
    mb_hi = (m_band0 + 2 * MAX_REL + 1 + 15) // 16

    def build_u(u, _):
        ut, us = u // 8, u % 8
        t0 = plsc.load_gather(tab_vmem, [lanes * 0 + u])
        t1 = plsc.load_gather(tab_vmem, [lanes * 0 + (2 * MAX_REL * NUM_UNITS + u)])

        def fill_head(mb, _):
            w_vmem[ut, us, pl.ds(mb * 16, 16)] = t0
            return 0

        def build_band(mb, _):
            m = mb * 16
            k = wof + m - (LK - 1) + lanes
            cidx = jnp.clip(k, -MAX_REL, MAX_REL) + MAX_REL
            w_vmem[ut, us, pl.ds(m, 16)] = plsc.load_gather(
                tab_vmem, [cidx * NUM_UNITS + u])
            return 0

        def fill_tail(mb, _):
            w_vmem[ut, us, pl.ds(mb * 16, 16)] = t1
            return 0

        lax.fori_loop(0, mb_lo, fill_head, 0)
        lax.fori_loop(mb_lo, mb_hi, build_band, 0)
        lax.fori_loop(mb_hi, W_COLS // 16, fill_tail, 0)
        return 0

    lax.fori_loop(0, NUM_UNITS, build_u, 0)

    def drain_row():
        for _ in range(JT):
            pltpu.make_async_copy(
                w_vmem.at[:, :, pl.ds(0, 128)],
                out_hbm.at[0, :, 0], sem).wait()

    def row(t, _):
        @pl.when(t > 1)
        def _():
            drain_row()
        i = phi + 512 * q + 8 * t
        s0 = 8 * (ROWS_PER_W - 1 - t)
        for jt in range(JT):
            pltpu.async_copy(
                w_vmem.at[:, :, pl.ds(s0 + 128 * jt, 128)],
                out_hbm.at[i, :, jt], sem)
        return 0

    lax.fori_loop(0, ROWS_PER_W, row, 0)
    for _ in range(2):
        drain_row()


def kernel(x, embeddings_table):
    del x
    run = pl.kernel(
        _sc_body,
        out_type=jax.ShapeDtypeStruct((LQ, UT, JT, 8, 128), jnp.float32),
        mesh=plsc.VectorSubcoreMesh(core_axis_name="c", subcore_axis_name="s"),
        scratch_types=[
            pltpu.VMEM((TAB_WORDS,), jnp.float32),
            pltpu.VMEM((UT, 8, W_COLS), jnp.float32),
            pltpu.SemaphoreType.DMA,
        ],
        compiler_params=pltpu.CompilerParams(use_tc_tiling_on_sc=False,
                                             needs_layout_passes=False),
    )
    out5 = run(embeddings_table.reshape(-1))
    return out5.transpose(0, 2, 4, 1, 3).reshape(LQ, LK, NUM_UNITS)
